# baseline (speedup 1.0000x reference)
import functools

import jax
import jax.numpy as jnp
from jax import lax
from jax.experimental import pallas as pl
from jax.experimental.pallas import tpu as pltpu
from jax.experimental.pallas import tpu_sc as plsc

_UPDATE_FACTOR = 0.6
_NUM_CLASSES = 1000000
_D = 64
_B = 16384

_BI = 256
_BJ = 2048


def _sc_mesh():
  return plsc.VectorSubcoreMesh(core_axis_name="c", subcore_axis_name="s")


_SC_PARAMS = pltpu.CompilerParams(needs_layout_passes=False)


@functools.cache
def _sc_geometry():
  info = plsc.get_sparse_core_info()
  nc = info.num_cores
  nw = nc * info.num_subcores
  assert _B % nw == 0 and (_B // nw) % 16 == 0
  return nc, nw, _B // nw


def _extract_scalar(lab_v, j):
  base = (j // 16) * 16
  v = lab_v[pl.ds(base, 16)]
  sel = jnp.where(lax.iota(jnp.int32, 16) == j - base, v, 0)
  return lax.reduce_max(sel, axes=(0,))


def _gather_rows(centers, labels2):
  nc, nw, p = _sc_geometry()

  @functools.partial(
      pl.kernel,
      mesh=_sc_mesh(),
      compiler_params=_SC_PARAMS,
      out_type=jax.ShapeDtypeStruct((nw, p, _D), jnp.float32),
      scratch_types=[
          pltpu.VMEM((p,), jnp.int32),
          pltpu.VMEM((p, _D), jnp.float32),
          pltpu.SemaphoreType.DMA,
      ],
  )
  def gather_kernel(centers_hbm, labels_hbm, out_hbm, lab_v, rows_v, sem):
    wid = lax.axis_index("s") * nc + lax.axis_index("c")
    pltpu.sync_copy(labels_hbm.at[wid], lab_v)

    def fire(j, carry):
      l = _extract_scalar(lab_v, j)
      pltpu.async_copy(
          centers_hbm.at[pl.ds(l, 1), :], rows_v.at[pl.ds(j, 1), :], sem
      )
      return carry

    lax.fori_loop(0, p, fire, 0)

    def drain(j, carry):
      pltpu.make_async_copy(
          centers_hbm.at[pl.ds(0, 1), :], rows_v.at[pl.ds(0, 1), :], sem
      ).wait()
      return carry

    lax.fori_loop(0, p, drain, 0)
    pltpu.sync_copy(rows_v, out_hbm.at[wid])

  return gather_kernel(centers, labels2)


def _scatter_rows(vals3, labels2, out_ref):
  nc, nw, p = _sc_geometry()

  @functools.partial(
      pl.kernel,
      mesh=_sc_mesh(),
      compiler_params=_SC_PARAMS,
      out_type=(),
      scratch_types=[
          pltpu.VMEM((p,), jnp.int32),
          pltpu.VMEM((p, _D), jnp.float32),
          pltpu.SemaphoreType.DMA,
      ],
  )
  def scatter_kernel(vals_hbm, labels_hbm, out_hbm, lab_v, rows_v, sem):
    wid = lax.axis_index("s") * nc + lax.axis_index("c")
    pltpu.sync_copy(labels_hbm.at[wid], lab_v)
    pltpu.sync_copy(vals_hbm.at[wid], rows_v)

    def fire(j, carry):
      l = _extract_scalar(lab_v, j)
      pltpu.async_copy(
          rows_v.at[pl.ds(j, 1), :], out_hbm.at[pl.ds(l, 1), :], sem
      )
      return carry

    lax.fori_loop(0, p, fire, 0)

    def drain(j, carry):
      pltpu.make_async_copy(
          rows_v.at[pl.ds(0, 1), :], out_hbm.at[pl.ds(0, 1), :], sem
      ).wait()
      return carry

    lax.fori_loop(0, p, drain, 0)

  scatter_kernel(vals3, labels2, out_ref)


_COPY_CHUNKS = 8


def _copy_into(centers, out_ref):
  mesh = pltpu.create_tensorcore_mesh("x")
  ncores = mesh.shape["x"]
  per = _NUM_CLASSES // ncores
  ch = per // _COPY_CHUNKS
  rem = _NUM_CLASSES - ncores * per

  @functools.partial(
      pl.kernel,
      mesh=mesh,
      out_type=(),
      scratch_types=[pltpu.SemaphoreType.DMA],
  )
  def copy_kernel(src_hbm, dst_hbm, sem):
    cid = lax.axis_index("x")
    base = cid * per
    copies = [
        pltpu.async_copy(
            src_hbm.at[pl.ds(base + k * ch, ch), :],
            dst_hbm.at[pl.ds(base + k * ch, ch), :],
            sem,
        )
        for k in range(_COPY_CHUNKS)
    ]
    tail = per - _COPY_CHUNKS * ch
    if tail:
      copies.append(
          pltpu.async_copy(
              src_hbm.at[pl.ds(base + _COPY_CHUNKS * ch, tail), :],
              dst_hbm.at[pl.ds(base + _COPY_CHUNKS * ch, tail), :],
              sem,
          )
      )
    if rem:
      @pl.when(cid == ncores - 1)
      def _():
        pltpu.async_copy(
            src_hbm.at[pl.ds(ncores * per, rem), :],
            dst_hbm.at[pl.ds(ncores * per, rem), :],
            sem,
        ).wait()
    for c in copies:
      c.wait()

  copy_kernel(centers, out_ref)


def _vals_kernel(lab_col_ref, lab_row_ref, embb_ref, embf_ref, g_ref, out_ref):
  li = lab_col_ref[...]

  def body(j, acc):
    sums, cnt = acc
    lj = lab_row_ref[:, pl.ds(j * _BJ, _BJ)]
    m = li == lj
    mb = m.astype(jnp.bfloat16)
    sums = sums + jnp.dot(
        mb, embb_ref[pl.ds(j * _BJ, _BJ), :], preferred_element_type=jnp.float32
    )
    cnt = cnt + jnp.sum(m.astype(jnp.float32), axis=1, keepdims=True)
    return sums, cnt

  sums0 = jnp.zeros((_BI, _D), jnp.float32)
  cnt0 = jnp.zeros((_BI, 1), jnp.float32)
  sums, cnt = lax.fori_loop(0, _B // _BJ, body, (sums0, cnt0))
  e = embf_ref[...]
  sums = sums - e.astype(jnp.bfloat16).astype(jnp.float32) + e
  g = g_ref[...]
  out_ref[...] = g + _UPDATE_FACTOR * (sums / cnt - g)


def _compute_vals(labels, emb, g):
  lab_col = labels.reshape(_B, 1)
  lab_row = labels.reshape(1, _B)
  embb = emb.astype(jnp.bfloat16)
  grid = (_B // _BI,)
  return pl.pallas_call(
      _vals_kernel,
      grid=grid,
      in_specs=[
          pl.BlockSpec((_BI, 1), lambda i: (i, 0)),
          pl.BlockSpec((1, _B), lambda i: (0, 0)),
          pl.BlockSpec((_B, _D), lambda i: (0, 0)),
          pl.BlockSpec((_BI, _D), lambda i: (i, 0)),
          pl.BlockSpec((_BI, _D), lambda i: (i, 0)),
      ],
      out_specs=pl.BlockSpec((_BI, _D), lambda i: (i, 0)),
      out_shape=jax.ShapeDtypeStruct((_B, _D), jnp.float32),
  )(lab_col, lab_row, embb, emb, g)


def kernel(embeddings, labels, centers):
  _, nw, p = _sc_geometry()
  labels = labels.astype(jnp.int32)
  labels2 = labels.reshape(nw, p)
  g = _gather_rows(centers, labels2).reshape(_B, _D)
  vals = _compute_vals(labels, embeddings, g)
  out_ref = jax.empty_ref(jax.ShapeDtypeStruct((_NUM_CLASSES, _D), jnp.float32))
  _copy_into(centers, out_ref)
  _scatter_rows(vals.reshape(nw, p, _D), labels2, out_ref)
  return jax.freeze(out_ref)


# speedup vs baseline: 11.8758x; 11.8758x over previous
import functools

import jax
import jax.numpy as jnp
from jax import lax
from jax.experimental import pallas as pl
from jax.experimental.pallas import tpu as pltpu
from jax.experimental.pallas import tpu_sc as plsc

_UPDATE_FACTOR = 0.6
_NUM_CLASSES = 1000000
_D = 64
_B = 16384

_BI = 256
_BJ = 2048


def _sc_mesh():
  return plsc.VectorSubcoreMesh(core_axis_name="c", subcore_axis_name="s")


_SC_PARAMS = pltpu.CompilerParams(needs_layout_passes=False)


@functools.cache
def _sc_geometry():
  info = plsc.get_sparse_core_info()
  nc = info.num_cores
  nw = nc * info.num_subcores
  assert _B % nw == 0 and (_B // nw) % 16 == 0
  return nc, nw, _B // nw


def _extract_scalar(lab_v, j):
  base = (j // 16) * 16
  v = lab_v[pl.ds(base, 16)]
  sel = jnp.where(lax.iota(jnp.int32, 16) == j - base, v, 0)
  return lax.reduce_max(sel, axes=(0,))


def _gather_rows(centers, labels2):
  nc, nw, p = _sc_geometry()

  @functools.partial(
      pl.kernel,
      mesh=_sc_mesh(),
      compiler_params=_SC_PARAMS,
      out_type=jax.ShapeDtypeStruct((nw, p, _D), jnp.float32),
      scratch_types=[
          pltpu.VMEM((p,), jnp.int32),
          pltpu.VMEM((p, _D), jnp.float32),
          pltpu.SemaphoreType.DMA,
      ],
  )
  def gather_kernel(centers_hbm, labels_hbm, out_hbm, lab_v, rows_v, sem):
    wid = lax.axis_index("s") * nc + lax.axis_index("c")
    pltpu.sync_copy(labels_hbm.at[wid], lab_v)

    def fire(j, carry):
      l = _extract_scalar(lab_v, j)
      pltpu.async_copy(
          centers_hbm.at[pl.ds(l, 1), :], rows_v.at[pl.ds(j, 1), :], sem
      )
      return carry

    lax.fori_loop(0, p, fire, 0)

    def drain(j, carry):
      pltpu.make_async_copy(
          centers_hbm.at[pl.ds(0, 1), :], rows_v.at[pl.ds(0, 1), :], sem
      ).wait()
      return carry

    lax.fori_loop(0, p, drain, 0)
    pltpu.sync_copy(rows_v, out_hbm.at[wid])

  return gather_kernel(centers, labels2)


def _scatter_rows(vals3, labels2, out_ref):
  nc, nw, p = _sc_geometry()

  @functools.partial(
      pl.kernel,
      mesh=_sc_mesh(),
      compiler_params=_SC_PARAMS,
      out_type=(),
      scratch_types=[
          pltpu.VMEM((p,), jnp.int32),
          pltpu.VMEM((p, _D), jnp.float32),
          pltpu.SemaphoreType.DMA,
      ],
  )
  def scatter_kernel(vals_hbm, labels_hbm, out_hbm, lab_v, rows_v, sem):
    wid = lax.axis_index("s") * nc + lax.axis_index("c")
    pltpu.sync_copy(labels_hbm.at[wid], lab_v)
    pltpu.sync_copy(vals_hbm.at[wid], rows_v)

    def fire(j, carry):
      l = _extract_scalar(lab_v, j)
      pltpu.async_copy(
          rows_v.at[pl.ds(j, 1), :], out_hbm.at[pl.ds(l, 1), :], sem
      )
      return carry

    lax.fori_loop(0, p, fire, 0)

    def drain(j, carry):
      pltpu.make_async_copy(
          rows_v.at[pl.ds(0, 1), :], out_hbm.at[pl.ds(0, 1), :], sem
      ).wait()
      return carry

    lax.fori_loop(0, p, drain, 0)

  scatter_kernel(vals3, labels2, out_ref)


_COPY_BLK = 8000


def _copy_body(src_ref, dst_ref):
  dst_ref[...] = src_ref[...]


def _copy_centers(centers):
  return pl.pallas_call(
      _copy_body,
      grid=(_NUM_CLASSES // _COPY_BLK,),
      in_specs=[pl.BlockSpec((_COPY_BLK, _D), lambda i: (i, 0))],
      out_specs=pl.BlockSpec((_COPY_BLK, _D), lambda i: (i, 0)),
      out_shape=jax.ShapeDtypeStruct((_NUM_CLASSES, _D), jnp.float32),
  )(centers)


def _vals_kernel(lab_col_ref, lab_row_ref, embb_ref, embf_ref, g_ref, out_ref):
  li = lab_col_ref[...]

  def body(j, acc):
    sums, cnt = acc
    lj = lab_row_ref[:, pl.ds(j * _BJ, _BJ)]
    m = li == lj
    mb = m.astype(jnp.bfloat16)
    sums = sums + jnp.dot(
        mb, embb_ref[pl.ds(j * _BJ, _BJ), :], preferred_element_type=jnp.float32
    )
    cnt = cnt + jnp.sum(m.astype(jnp.float32), axis=1, keepdims=True)
    return sums, cnt

  sums0 = jnp.zeros((_BI, _D), jnp.float32)
  cnt0 = jnp.zeros((_BI, 1), jnp.float32)
  sums, cnt = lax.fori_loop(0, _B // _BJ, body, (sums0, cnt0))
  e = embf_ref[...]
  sums = sums - e.astype(jnp.bfloat16).astype(jnp.float32) + e
  g = g_ref[...]
  out_ref[...] = g + _UPDATE_FACTOR * (sums / cnt - g)


def _compute_vals(labels, emb, g):
  lab_col = labels.reshape(_B, 1)
  lab_row = labels.reshape(1, _B)
  embb = emb.astype(jnp.bfloat16)
  grid = (_B // _BI,)
  return pl.pallas_call(
      _vals_kernel,
      grid=grid,
      in_specs=[
          pl.BlockSpec((_BI, 1), lambda i: (i, 0)),
          pl.BlockSpec((1, _B), lambda i: (0, 0)),
          pl.BlockSpec((_B, _D), lambda i: (0, 0)),
          pl.BlockSpec((_BI, _D), lambda i: (i, 0)),
          pl.BlockSpec((_BI, _D), lambda i: (i, 0)),
      ],
      out_specs=pl.BlockSpec((_BI, _D), lambda i: (i, 0)),
      out_shape=jax.ShapeDtypeStruct((_B, _D), jnp.float32),
  )(lab_col, lab_row, embb, emb, g)


def kernel(embeddings, labels, centers):
  _, nw, p = _sc_geometry()
  labels = labels.astype(jnp.int32)
  labels2 = labels.reshape(nw, p)
  g = _gather_rows(centers, labels2).reshape(_B, _D)
  vals = _compute_vals(labels, embeddings, g)
  out_ref = jax.new_ref(_copy_centers(centers))
  _scatter_rows(vals.reshape(nw, p, _D), labels2, out_ref)
  return jax.freeze(out_ref)
